# baseline (speedup 1.0000x reference)
import jax
import jax.numpy as jnp
from jax.experimental import pallas as pl

_K = 1000
_N_BISECT = 19


def _mask_kernel(data_ref, wt_ref, b_ref, out_ref):
    x = data_ref[...]
    w = wt_ref[...]
    out = jnp.dot(x, w, preferred_element_type=jnp.float32) + b_ref[...]

    rmax = jnp.max(out, axis=1, keepdims=True)
    rmin = jnp.min(out, axis=1, keepdims=True)
    span = rmax - rmin
    lo0 = rmin - (span * 1e-3 + 1e-30)
    hi0 = rmax

    lo, hi = lo0, hi0
    for _ in range(_N_BISECT):
        mid = 0.5 * (lo + hi)
        cnt = jnp.sum((out > mid).astype(jnp.float32), axis=1, keepdims=True)
        pred = cnt >= _K
        lo = jnp.where(pred, mid, lo)
        hi = jnp.where(pred, hi, mid)
    out_ref[...] = jnp.where(out > lo, out, 0.0)


def kernel(data, W, b):
    B, D = data.shape
    V = W.shape[0]
    R = 8 if B % 8 == 0 else B
    wt = W.T
    b2 = b.reshape(1, V)
    return pl.pallas_call(
        _mask_kernel,
        grid=(B // R,),
        in_specs=[
            pl.BlockSpec((R, D), lambda i: (i, 0)),
            pl.BlockSpec((D, V), lambda i: (0, 0)),
            pl.BlockSpec((1, V), lambda i: (0, 0)),
        ],
        out_specs=pl.BlockSpec((R, V), lambda i: (i, 0)),
        out_shape=jax.ShapeDtypeStruct((B, V), jnp.float32),
    )(data, wt, b2)


# speedup vs baseline: 1.4710x; 1.4710x over previous
import jax
import jax.numpy as jnp
from jax.experimental import pallas as pl

_K = 1000
_N_BISECT = 19


def _bisect_init(out):
    rmax = jnp.max(out, axis=1, keepdims=True)
    rmin = jnp.min(out, axis=1, keepdims=True)
    span = rmax - rmin
    lo = rmin - (span * 1e-3 + 1e-30)
    return lo, rmax


def _bisect_step(out, lo, hi):
    mid = 0.5 * (lo + hi)
    cnt = jnp.sum((out > mid).astype(jnp.float32), axis=1, keepdims=True)
    pred = cnt >= _K
    return jnp.where(pred, mid, lo), jnp.where(pred, hi, mid)


def _mask_kernel(data_ref, wt_ref, b_ref, out_ref):
    x = data_ref[...]
    w = wt_ref[...]
    bb = b_ref[...]
    h = x.shape[0] // 2
    out_a = jnp.dot(x[:h], w, preferred_element_type=jnp.float32) + bb
    out_b = jnp.dot(x[h:], w, preferred_element_type=jnp.float32) + bb

    lo_a, hi_a = _bisect_init(out_a)
    lo_b, hi_b = _bisect_init(out_b)
    for _ in range(_N_BISECT):
        lo_a, hi_a = _bisect_step(out_a, lo_a, hi_a)
        lo_b, hi_b = _bisect_step(out_b, lo_b, hi_b)
    out_ref[:h] = jnp.where(out_a > lo_a, out_a, 0.0)
    out_ref[h:] = jnp.where(out_b > lo_b, out_b, 0.0)


def kernel(data, W, b):
    B, D = data.shape
    V = W.shape[0]
    R = 32 if B % 32 == 0 else (16 if B % 16 == 0 else B)
    wt = W.T
    b2 = b.reshape(1, V)
    return pl.pallas_call(
        _mask_kernel,
        grid=(B // R,),
        in_specs=[
            pl.BlockSpec((R, D), lambda i: (i, 0)),
            pl.BlockSpec((D, V), lambda i: (0, 0)),
            pl.BlockSpec((1, V), lambda i: (0, 0)),
        ],
        out_specs=pl.BlockSpec((R, V), lambda i: (i, 0)),
        out_shape=jax.ShapeDtypeStruct((B, V), jnp.float32),
    )(data, wt, b2)


# speedup vs baseline: 1.9662x; 1.3366x over previous
import jax
import jax.numpy as jnp
from jax.experimental import pallas as pl

_K = 1000
_N_BISECT = 19


def _mask_kernel(data_ref, wt_ref, b_ref, out_ref):
    x = data_ref[...]
    w = wt_ref[...]
    xh = x.astype(jnp.bfloat16)
    xl = (x - xh.astype(jnp.float32)).astype(jnp.bfloat16)
    wh = w.astype(jnp.bfloat16)
    wl = (w - wh.astype(jnp.float32)).astype(jnp.bfloat16)
    out = (jnp.dot(xh, wh, preferred_element_type=jnp.float32)
           + jnp.dot(xh, wl, preferred_element_type=jnp.float32)
           + jnp.dot(xl, wh, preferred_element_type=jnp.float32)
           + b_ref[...])

    rmax = jnp.max(out, axis=1, keepdims=True)
    rmin = jnp.min(out, axis=1, keepdims=True)
    span = rmax - rmin
    lo0 = rmin - (span * 1e-3 + 1e-30)
    hi0 = rmax

    lo, hi = lo0, hi0
    for _ in range(_N_BISECT):
        mid = 0.5 * (lo + hi)
        cnt = jnp.sum((out > mid).astype(jnp.float32), axis=1, keepdims=True)
        pred = cnt >= _K
        lo = jnp.where(pred, mid, lo)
        hi = jnp.where(pred, hi, mid)
    out_ref[...] = jnp.where(out > lo, out, 0.0)


def kernel(data, W, b):
    B, D = data.shape
    V = W.shape[0]
    R = 32 if B % 32 == 0 else (8 if B % 8 == 0 else B)
    wt = W.T
    b2 = b.reshape(1, V)
    return pl.pallas_call(
        _mask_kernel,
        grid=(B // R,),
        in_specs=[
            pl.BlockSpec((R, D), lambda i: (i, 0)),
            pl.BlockSpec((D, V), lambda i: (0, 0)),
            pl.BlockSpec((1, V), lambda i: (0, 0)),
        ],
        out_specs=pl.BlockSpec((R, V), lambda i: (i, 0)),
        out_shape=jax.ShapeDtypeStruct((B, V), jnp.float32),
    )(data, wt, b2)
